# baseline (speedup 1.0000x reference)
import functools

import jax
import jax.numpy as jnp
from jax import lax
from jax.experimental import pallas as pl
from jax.experimental.pallas import tpu as pltpu
from jax.experimental.pallas import tpu_sc as plsc

TB = 256
NC, NS, L = 2, 16, 16
NW = NC * NS


def _router_body(x_ref, wr_ref, scores_ref, w_ref, i_ref, rank_ref,
                 counts_ref, run_ref, *, n_e, n_t):
    t = pl.program_id(0)
    tb = x_ref.shape[0]
    logits = jnp.dot(x_ref[...], wr_ref[...], preferred_element_type=jnp.float32)
    m = jnp.max(logits, axis=-1, keepdims=True)
    ex = jnp.exp(logits - m)
    sc = ex / jnp.sum(ex, axis=-1, keepdims=True)
    scores_ref[...] = sc
    eio = lax.broadcasted_iota(jnp.int32, sc.shape, 1)
    w0 = jnp.max(sc, axis=-1, keepdims=True)
    i0 = jnp.min(jnp.where(sc == w0, eio, n_e), axis=-1, keepdims=True)
    sc2 = jnp.where(eio == i0, -1.0, sc)
    w1 = jnp.max(sc2, axis=-1, keepdims=True)
    i1 = jnp.min(jnp.where(sc2 == w1, eio, n_e), axis=-1, keepdims=True)
    ssum = w0 + w1
    w0n = w0 / ssum
    w1n = w1 / ssum
    lane2 = lax.broadcasted_iota(jnp.int32, (tb, 2), 1)
    w_ref[...] = jnp.where(lane2 == 0, w0n, w1n)
    i_ref[...] = jnp.where(lane2 == 0, i0, i1)
    oh0 = (eio == i0).astype(jnp.float32)
    oh1 = (eio == i1).astype(jnp.float32)
    oh = oh0 + oh1
    r_iota = lax.broadcasted_iota(jnp.int32, (tb, tb), 0)
    c_iota = lax.broadcasted_iota(jnp.int32, (tb, tb), 1)
    tri = (c_iota < r_iota).astype(jnp.float32)
    base = jnp.where(t == 0, jnp.zeros_like(run_ref), run_ref[...])
    cum = base + jnp.dot(tri, oh, preferred_element_type=jnp.float32)
    rank0 = jnp.sum(oh0 * cum, axis=1, keepdims=True)
    rank1 = jnp.sum(oh1 * cum, axis=1, keepdims=True)
    rank_ref[...] = jnp.where(lane2 == 0, rank0, rank1).astype(jnp.int32)
    newrun = base + jnp.sum(oh, axis=0, keepdims=True)
    run_ref[...] = newrun

    @pl.when(t == n_t - 1)
    def _():
        cnt16 = jnp.concatenate(
            [newrun, jnp.zeros((1, L - n_e), jnp.float32)], axis=1)
        padded = jnp.floor((cnt16 + (TB - 1)) / TB) * TB
        a16 = lax.broadcasted_iota(jnp.int32, (L, L), 0)
        b16 = lax.broadcasted_iota(jnp.int32, (L, L), 1)
        tri_incl = (a16 <= b16).astype(jnp.float32)
        ends = jnp.dot(padded, tri_incl, preferred_element_type=jnp.float32)
        off = ends - padded
        counts_ref[...] = jnp.concatenate([off, ends], axis=0).astype(jnp.int32)


def _router(x_flat, W_router):
    n_tok, d = x_flat.shape
    n_e = W_router.shape[1]
    n_t = n_tok // TB
    return pl.pallas_call(
        functools.partial(_router_body, n_e=n_e, n_t=n_t),
        grid=(n_t,),
        in_specs=[pl.BlockSpec((TB, d), lambda i: (i, 0)),
                  pl.BlockSpec((d, n_e), lambda i: (0, 0))],
        out_specs=[pl.BlockSpec((TB, n_e), lambda i: (i, 0)),
                   pl.BlockSpec((TB, 2), lambda i: (i, 0)),
                   pl.BlockSpec((TB, 2), lambda i: (i, 0)),
                   pl.BlockSpec((TB, 2), lambda i: (i, 0)),
                   pl.BlockSpec((2, L), lambda i: (0, 0))],
        out_shape=[jax.ShapeDtypeStruct((n_tok, n_e), jnp.float32),
                   jax.ShapeDtypeStruct((n_tok, 2), jnp.float32),
                   jax.ShapeDtypeStruct((n_tok, 2), jnp.int32),
                   jax.ShapeDtypeStruct((n_tok, 2), jnp.int32),
                   jax.ShapeDtypeStruct((2, L), jnp.int32)],
        scratch_shapes=[pltpu.VMEM((1, n_e), jnp.float32)],
        compiler_params=pltpu.CompilerParams(dimension_semantics=("arbitrary",)),
    )(x_flat, W_router)


def _make_dispatch(n_assign, padn, nbpad, n_e, d):
    per_w = n_assign // NW
    per_s = n_assign // NS
    zchunk = padn // NS
    nb_chunks = nbpad // L
    g_per_w = padn // NW
    ghalf = g_per_w // 2
    gch = 32
    dp = d // 2
    ngch = ghalf // gch
    mesh = plsc.VectorSubcoreMesh(core_axis_name="c", subcore_axis_name="s")

    @functools.partial(
        pl.kernel,
        out_type=[jax.ShapeDtypeStruct((n_assign,), jnp.int32),
                  jax.ShapeDtypeStruct((padn,), jnp.float32),
                  jax.ShapeDtypeStruct((nbpad,), jnp.int32),
                  jax.ShapeDtypeStruct((L,), jnp.int32),
                  jax.ShapeDtypeStruct((padn, dp), jnp.int32)],
        mesh=mesh,
        scratch_types=[
            pltpu.VMEM((2, L), jnp.int32),
            pltpu.VMEM((per_w,), jnp.int32),
            pltpu.VMEM((per_w,), jnp.int32),
            pltpu.VMEM((per_w,), jnp.int32),
            pltpu.VMEM((per_s,), jnp.int32),
            pltpu.VMEM((per_s,), jnp.int32),
            pltpu.VMEM((per_s,), jnp.float32),
            pltpu.VMEM((per_s // 128, 128), jnp.int32),
            pltpu.VMEM((per_s // 128, 128), jnp.int32),
            pltpu.VMEM((per_s // 128, 128), jnp.float32),
            pltpu.VMEM((zchunk,), jnp.int32),
            pltpu.VMEM((zchunk,), jnp.float32),
            pltpu.VMEM((nbpad,), jnp.int32),
            pltpu.VMEM((L,), jnp.int32),
            pltpu.VMEM((g_per_w,), jnp.int32),
            pltpu.VMEM((gch, dp), jnp.int32),
            pltpu.VMEM((gch, dp), jnp.int32),
            pltpu.VMEM((gch, dp), jnp.int32),
            pltpu.VMEM((gch, dp), jnp.int32),
            pltpu.SemaphoreType.DMA,
            pltpu.SemaphoreType.DMA,
            pltpu.SemaphoreType.DMA,
            pltpu.SemaphoreType.DMA,
            pltpu.VMEM_SHARED((padn,), jnp.int32),
            pltpu.VMEM_SHARED((padn,), jnp.float32),
        ],
    )
    def dispatch(idx_hbm, rank_hbm, w_hbm, meta_hbm, x_hbm,
                 pos_hbm, dw_hbm, bexp_hbm, nblk_hbm, xg_hbm,
                 meta_v, a_idx, a_rank, posbuf,
                 b_idx, b_rank, b_w, sc_idx, sc_tok, sc_w,
                 zero_i, zero_f, bexp_v, nblk_v,
                 gidx_v, ga0, ga1, gb0, gb1, gsa0, gsa1, gsb0, gsb1,
                 sh_tok, sh_w):
        c = lax.axis_index("c")
        s = lax.axis_index("s")
        wid = s * NC + c
        lane = lax.iota(jnp.int32, L)

        pltpu.sync_copy(meta_hbm, meta_v)
        n_tok_mask = n_assign // 2 - 1
        off_row = meta_v[0]
        ends_row = meta_v[1]
        offs = [off_row[e] for e in range(n_e)]
        ends_sc = [ends_row[e] for e in range(n_e)]

        def sel_off(ev):
            acc = jnp.zeros((L,), jnp.int32)
            for e in range(n_e):
                acc = jnp.where(ev == e, offs[e], acc)
            return acc

        base = wid * per_w
        pltpu.sync_copy(idx_hbm.at[pl.ds(base, per_w)], a_idx)
        pltpu.sync_copy(rank_hbm.at[pl.ds(base, per_w)], a_rank)
        for k in range(per_w // L):
            ev = a_idx[pl.ds(k * L, L)]
            posbuf[pl.ds(k * L, L)] = a_rank[pl.ds(k * L, L)] + sel_off(ev)
        pltpu.sync_copy(posbuf, pos_hbm.at[pl.ds(base, per_w)])

        for k in range(zchunk // L):
            zero_i[pl.ds(k * L, L)] = jnp.zeros((L,), jnp.int32)
            zero_f[pl.ds(k * L, L)] = jnp.zeros((L,), jnp.float32)
        pltpu.sync_copy(zero_i, sh_tok.at[pl.ds(s * zchunk, zchunk)])
        pltpu.sync_copy(zero_f, sh_w.at[pl.ds(s * zchunk, zchunk)])
        plsc.subcore_barrier()

        sbase = s * per_s
        pltpu.sync_copy(idx_hbm.at[pl.ds(sbase, per_s)], b_idx)
        pltpu.sync_copy(rank_hbm.at[pl.ds(sbase, per_s)], b_rank)
        pltpu.sync_copy(w_hbm.at[pl.ds(sbase, per_s)], b_w)
        for q in range(per_s // 128):
            for k in range(128 // L):
                j0 = q * 128 + k * L
                ev = b_idx[pl.ds(j0, L)]
                sc_idx[q, pl.ds(k * L, L)] = b_rank[pl.ds(j0, L)] + sel_off(ev)
                sc_tok[q, pl.ds(k * L, L)] = (sbase + j0 + lane) & n_tok_mask
                sc_w[q, pl.ds(k * L, L)] = b_w[pl.ds(j0, L)]

        for q in range(per_s // 128):
            pltpu.sync_copy(sc_tok.at[q], sh_tok.at[sc_idx.at[q]], add=True)

        @pl.when(c == 1)
        def _():
            for q in range(per_s // 128):
                pltpu.sync_copy(sc_w.at[q], sh_w.at[sc_idx.at[q]], add=True)

        plsc.subcore_barrier()

        @pl.when((s == 0) & (c == 1))
        def _():
            pltpu.sync_copy(sh_w, dw_hbm)

        @pl.when((s == 1) & (c == 0))
        def _():
            for ch in range(nb_chunks):
                bstart = (ch * L + lane) * TB
                acc = jnp.zeros((L,), jnp.int32)
                for e in range(n_e):
                    acc = acc + jnp.where(bstart >= ends_sc[e], 1, 0)
                bexp_v[pl.ds(ch * L, L)] = jnp.minimum(acc, n_e - 1)
            pltpu.sync_copy(bexp_v, bexp_hbm)
            tb_sh = TB.bit_length() - 1
            nreal = (ends_sc[n_e - 1] + (TB - 1)) >> tb_sh
            nblk_v[...] = jnp.where(lane == 0, nreal, 0)
            pltpu.sync_copy(nblk_v, nblk_hbm)

        gbase = wid * g_per_w
        pltpu.sync_copy(sh_tok.at[pl.ds(gbase, g_per_w)], gidx_v)
        abuf = (ga0, ga1)
        bbuf = (gb0, gb1)
        asem = (gsa0, gsa1)
        bsem = (gsb0, gsb1)

        def fire_a(k):
            return pltpu.async_copy(
                x_hbm.at[gidx_v.at[pl.ds(k * gch, gch)]],
                abuf[k % 2], asem[k % 2])

        def fire_b(k):
            return pltpu.async_copy(
                x_hbm.at[gidx_v.at[pl.ds(ghalf + k * gch, gch)]],
                bbuf[k % 2], bsem[k % 2])

        acp = [None] * ngch
        bcp = [None] * ngch
        acp[0] = fire_a(0)
        bcp[0] = fire_b(0)
        for k in range(ngch):
            acp[k].wait()
            if k + 1 < ngch:
                acp[k + 1] = fire_a(k + 1)
            pltpu.sync_copy(abuf[k % 2], xg_hbm.at[pl.ds(gbase + k * gch, gch)])
            bcp[k].wait()
            if k + 1 < ngch:
                bcp[k + 1] = fire_b(k + 1)
            pltpu.sync_copy(bbuf[k % 2],
                            xg_hbm.at[pl.ds(gbase + ghalf + k * gch, gch)])

    return dispatch


def _make_gather(padn, d, dt):
    per_w = padn // NW
    ch = 40 if jnp.dtype(dt).itemsize == 4 else 64
    nch = per_w // ch
    mesh = plsc.VectorSubcoreMesh(core_axis_name="c", subcore_axis_name="s")

    @functools.partial(
        pl.kernel,
        out_type=jax.ShapeDtypeStruct((padn, d), dt),
        mesh=mesh,
        scratch_types=[
            pltpu.VMEM((per_w,), jnp.int32),
            pltpu.VMEM((ch, d), dt),
            pltpu.VMEM((ch, d), dt),
            pltpu.SemaphoreType.DMA,
            pltpu.SemaphoreType.DMA,
        ],
    )
    def gather(x_hbm, dtok_hbm, xg_hbm, idx_v, rows0, rows1, sem0, sem1):
        c = lax.axis_index("c")
        s = lax.axis_index("s")
        base = (s * NC + c) * per_w
        pltpu.sync_copy(dtok_hbm.at[pl.ds(base, per_w)], idx_v)
        bufs = (rows0, rows1)
        sems = (sem0, sem1)
        cps = [None] * nch
        cps[0] = pltpu.async_copy(
            x_hbm.at[idx_v.at[pl.ds(0, ch)]], bufs[0], sems[0])
        for k in range(nch):
            cps[k].wait()
            if k + 1 < nch:
                cps[k + 1] = pltpu.async_copy(
                    x_hbm.at[idx_v.at[pl.ds((k + 1) * ch, ch)]],
                    bufs[(k + 1) % 2], sems[(k + 1) % 2])
            pltpu.sync_copy(bufs[k % 2], xg_hbm.at[pl.ds(base + k * ch, ch)])

    return gather


def _expert_body_first(bexp_ref, nblk_ref, xg_ref, wg_ref, wu_ref, wd_ref,
                       dw_ref, od_ref):
    @pl.when(pl.program_id(0) < nblk_ref[0])
    def _():
        xb = xg_ref[...].astype(jnp.float32)
        g = jnp.dot(xb, wg_ref[0], preferred_element_type=jnp.float32)
        u = jnp.dot(xb, wu_ref[0], preferred_element_type=jnp.float32)
        h = g * jax.nn.sigmoid(g) * u
        part = jnp.dot(h, wd_ref[0], preferred_element_type=jnp.float32) * dw_ref[...]
        od_ref[...] = part.astype(od_ref.dtype)


def _expert_body_second(bexp_ref, nblk_ref, xg_ref, wg_ref, wu_ref, wd_ref,
                        dw_ref, acc_ref, od_ref):
    @pl.when(pl.program_id(0) < nblk_ref[0])
    def _():
        xb = xg_ref[...].astype(jnp.float32)
        g = jnp.dot(xb, wg_ref[0], preferred_element_type=jnp.float32)
        u = jnp.dot(xb, wu_ref[0], preferred_element_type=jnp.float32)
        h = g * jax.nn.sigmoid(g) * u
        part = jnp.dot(h, wd_ref[0], preferred_element_type=jnp.float32) * dw_ref[...]
        od_ref[...] = (acc_ref[...].astype(jnp.float32) + part).astype(od_ref.dtype)


def _experts(bexp, nblk, xg, W_gate, W_up, W_down, dw2d, nb, od_dt):
    n_e, d, ff = W_gate.shape
    padn = xg.shape[0]
    fhalf = ff // 2

    def specs(fi, extra):
        return [
            pl.BlockSpec((TB, d), lambda b, be, nr: (b, 0)),
            pl.BlockSpec((1, d, fhalf), lambda b, be, nr: (be[b], 0, fi)),
            pl.BlockSpec((1, d, fhalf), lambda b, be, nr: (be[b], 0, fi)),
            pl.BlockSpec((1, fhalf, d), lambda b, be, nr: (be[b], fi, 0)),
            pl.BlockSpec((TB, 1), lambda b, be, nr: (b, 0)),
        ] + extra

    od_shape = jax.ShapeDtypeStruct((padn, d), od_dt)
    out_spec = pl.BlockSpec((TB, d), lambda b, be, nr: (b, 0))
    params = pltpu.CompilerParams(dimension_semantics=("arbitrary",))

    acc = pl.pallas_call(
        _expert_body_first,
        grid_spec=pltpu.PrefetchScalarGridSpec(
            num_scalar_prefetch=2, grid=(nb,),
            in_specs=specs(0, []), out_specs=out_spec),
        out_shape=od_shape,
        compiler_params=params,
    )(bexp, nblk, xg, W_gate, W_up, W_down, dw2d)

    return pl.pallas_call(
        _expert_body_second,
        grid_spec=pltpu.PrefetchScalarGridSpec(
            num_scalar_prefetch=2, grid=(nb,),
            in_specs=specs(1, [pl.BlockSpec((TB, d), lambda b, be, nr: (b, 0))]),
            out_specs=out_spec),
        out_shape=od_shape,
        compiler_params=params,
    )(bexp, nblk, xg, W_gate, W_up, W_down, dw2d, acc)


def _make_combine(n_tok, padn, d, dt):
    tok_per_w = n_tok // NW
    ch = 16
    nch = tok_per_w // ch
    lanes = 32 if jnp.dtype(dt).itemsize == 2 else 16
    mesh = plsc.VectorSubcoreMesh(core_axis_name="c", subcore_axis_name="s")

    @functools.partial(
        pl.kernel,
        out_type=jax.ShapeDtypeStruct((n_tok, d), dt),
        mesh=mesh,
        scratch_types=[
            pltpu.VMEM((tok_per_w,), jnp.int32),
            pltpu.VMEM((tok_per_w,), jnp.int32),
            pltpu.VMEM((ch, d), dt),
            pltpu.VMEM((ch, d), dt),
            pltpu.VMEM((ch, d), dt),
            pltpu.VMEM((ch, d), dt),
            pltpu.SemaphoreType.DMA,
            pltpu.SemaphoreType.DMA,
            pltpu.SemaphoreType.DMA,
            pltpu.SemaphoreType.DMA,
        ],
    )
    def combine(od_hbm, pos_hbm, out_hbm,
                pos0_v, pos1_v, a0, b0, a1, b1, sa0, sb0, sa1, sb1):
        c = lax.axis_index("c")
        s = lax.axis_index("s")
        wid = s * NC + c
        tbase = wid * tok_per_w
        n_tok_half = pos_hbm.shape[0] // 2
        n_col = d // lanes
        pltpu.sync_copy(pos_hbm.at[pl.ds(tbase, tok_per_w)], pos0_v)
        pltpu.sync_copy(pos_hbm.at[pl.ds(n_tok_half + tbase, tok_per_w)], pos1_v)
        abuf = (a0, a1)
        bbuf = (b0, b1)
        asem = (sa0, sa1)
        bsem = (sb0, sb1)

        def fire(k):
            p = k % 2
            return (pltpu.async_copy(
                        od_hbm.at[pos0_v.at[pl.ds(k * ch, ch)]], abuf[p], asem[p]),
                    pltpu.async_copy(
                        od_hbm.at[pos1_v.at[pl.ds(k * ch, ch)]], bbuf[p], bsem[p]))

        cps = [None] * nch
        cps[0] = fire(0)
        for k in range(nch):
            p = k % 2
            cps[k][0].wait()
            cps[k][1].wait()
            if k + 1 < nch:
                cps[k + 1] = fire(k + 1)
            ra, rb = abuf[p], bbuf[p]

            def addb(i, carry):
                r = i // n_col
                col = (i % n_col) * lanes
                ra[r, pl.ds(col, lanes)] = (ra[r, pl.ds(col, lanes)]
                                            + rb[r, pl.ds(col, lanes)])
                return carry

            lax.fori_loop(0, ch * n_col, addb, 0)
            pltpu.sync_copy(ra, out_hbm.at[pl.ds(tbase + k * ch, ch)])

    return combine


def kernel_routed(x, W_router, W_gate, W_up, W_down):
    b, t, d = x.shape
    n_tok = b * t
    n_e = W_router.shape[1]
    n_assign = 2 * n_tok
    padn = n_assign + n_e * TB
    nb = padn // TB
    nbpad = ((nb + L - 1) // L) * L
    x_flat = x.reshape(n_tok, d)

    scores, weights, indices, rank, meta = _router(x_flat, W_router)
    idx8 = indices.T.reshape(n_assign)
    rank8 = rank.T.reshape(n_assign)
    w8 = weights.T.reshape(n_assign)

    x_pk = lax.bitcast_convert_type(
        x_flat.astype(jnp.bfloat16).reshape(n_tok, d // 2, 2), jnp.int32)
    pos, dw, bexp, nblk, xg_pk = _make_dispatch(n_assign, padn, nbpad, n_e, d)(
        idx8, rank8, w8, meta, x_pk)
    xg = lax.bitcast_convert_type(xg_pk, jnp.bfloat16).reshape(padn, d)
    od = _experts(bexp, nblk, xg, W_gate, W_up, W_down,
                  dw.reshape(padn, 1), nb, jnp.float32)
    out_flat = _make_combine(n_tok, padn, d, jnp.float32)(od, pos)
    return out_flat.reshape(b, t, d), weights, indices, scores


kernel = kernel_routed


# speedup vs baseline: 1.5241x; 1.5241x over previous
import functools

import jax
import jax.numpy as jnp
from jax import lax
from jax.experimental import pallas as pl
from jax.experimental.pallas import tpu as pltpu
from jax.experimental.pallas import tpu_sc as plsc

TB = 256
NC, NS, L = 2, 16, 16
NW = NC * NS


def _router_body(x_ref, wr_ref, scores_ref, w_ref, i_ref, rank_ref,
                 counts_ref, run_ref, *, n_e, n_t):
    t = pl.program_id(0)
    tb = x_ref.shape[0]
    logits = jnp.dot(x_ref[...], wr_ref[...], preferred_element_type=jnp.float32)
    m = jnp.max(logits, axis=-1, keepdims=True)
    ex = jnp.exp(logits - m)
    sc = ex / jnp.sum(ex, axis=-1, keepdims=True)
    scores_ref[...] = sc
    eio = lax.broadcasted_iota(jnp.int32, sc.shape, 1)
    w0 = jnp.max(sc, axis=-1, keepdims=True)
    i0 = jnp.min(jnp.where(sc == w0, eio, n_e), axis=-1, keepdims=True)
    sc2 = jnp.where(eio == i0, -1.0, sc)
    w1 = jnp.max(sc2, axis=-1, keepdims=True)
    i1 = jnp.min(jnp.where(sc2 == w1, eio, n_e), axis=-1, keepdims=True)
    ssum = w0 + w1
    w0n = w0 / ssum
    w1n = w1 / ssum
    lane2 = lax.broadcasted_iota(jnp.int32, (tb, 2), 1)
    w_ref[...] = jnp.where(lane2 == 0, w0n, w1n)
    i_ref[...] = jnp.where(lane2 == 0, i0, i1)
    oh0 = (eio == i0).astype(jnp.float32)
    oh1 = (eio == i1).astype(jnp.float32)
    oh = oh0 + oh1
    r_iota = lax.broadcasted_iota(jnp.int32, (tb, tb), 0)
    c_iota = lax.broadcasted_iota(jnp.int32, (tb, tb), 1)
    tri = (c_iota < r_iota).astype(jnp.float32)
    base = jnp.where(t == 0, jnp.zeros_like(run_ref), run_ref[...])
    cum = base + jnp.dot(tri, oh, preferred_element_type=jnp.float32)
    rank0 = jnp.sum(oh0 * cum, axis=1, keepdims=True)
    rank1 = jnp.sum(oh1 * cum, axis=1, keepdims=True)
    rank_ref[...] = jnp.where(lane2 == 0, rank0, rank1).astype(jnp.int32)
    newrun = base + jnp.sum(oh, axis=0, keepdims=True)
    run_ref[...] = newrun

    @pl.when(t == n_t - 1)
    def _():
        cnt16 = jnp.concatenate(
            [newrun, jnp.zeros((1, L - n_e), jnp.float32)], axis=1)
        padded = jnp.floor((cnt16 + (TB - 1)) / TB) * TB
        a16 = lax.broadcasted_iota(jnp.int32, (L, L), 0)
        b16 = lax.broadcasted_iota(jnp.int32, (L, L), 1)
        tri_incl = (a16 <= b16).astype(jnp.float32)
        ends = jnp.dot(padded, tri_incl, preferred_element_type=jnp.float32)
        off = ends - padded
        counts_ref[...] = jnp.concatenate([off, ends], axis=0).astype(jnp.int32)


def _router(x_flat, W_router):
    n_tok, d = x_flat.shape
    n_e = W_router.shape[1]
    n_t = n_tok // TB
    return pl.pallas_call(
        functools.partial(_router_body, n_e=n_e, n_t=n_t),
        grid=(n_t,),
        in_specs=[pl.BlockSpec((TB, d), lambda i: (i, 0)),
                  pl.BlockSpec((d, n_e), lambda i: (0, 0))],
        out_specs=[pl.BlockSpec((TB, n_e), lambda i: (i, 0)),
                   pl.BlockSpec((TB, 2), lambda i: (i, 0)),
                   pl.BlockSpec((TB, 2), lambda i: (i, 0)),
                   pl.BlockSpec((TB, 2), lambda i: (i, 0)),
                   pl.BlockSpec((2, L), lambda i: (0, 0))],
        out_shape=[jax.ShapeDtypeStruct((n_tok, n_e), jnp.float32),
                   jax.ShapeDtypeStruct((n_tok, 2), jnp.float32),
                   jax.ShapeDtypeStruct((n_tok, 2), jnp.int32),
                   jax.ShapeDtypeStruct((n_tok, 2), jnp.int32),
                   jax.ShapeDtypeStruct((2, L), jnp.int32)],
        scratch_shapes=[pltpu.VMEM((1, n_e), jnp.float32)],
        compiler_params=pltpu.CompilerParams(dimension_semantics=("arbitrary",)),
    )(x_flat, W_router)


def _make_dispatch(n_assign, padn, nbpad, n_e, d):
    per_w = n_assign // NW
    per_s = n_assign // NS
    zchunk = padn // NS
    nb_chunks = nbpad // L
    g_per_w = padn // NW
    ghalf = g_per_w // 2
    gch = 16
    ngch = ghalf // gch
    mesh = plsc.VectorSubcoreMesh(core_axis_name="c", subcore_axis_name="s")

    @functools.partial(
        pl.kernel,
        out_type=[jax.ShapeDtypeStruct((n_assign,), jnp.int32),
                  jax.ShapeDtypeStruct((padn,), jnp.float32),
                  jax.ShapeDtypeStruct((nbpad,), jnp.int32),
                  jax.ShapeDtypeStruct((L,), jnp.int32),
                  jax.ShapeDtypeStruct((padn, d), jnp.float32)],
        mesh=mesh,
        scratch_types=[
            pltpu.VMEM((2, L), jnp.int32),
            pltpu.VMEM((per_w,), jnp.int32),
            pltpu.VMEM((per_w,), jnp.int32),
            pltpu.VMEM((per_w,), jnp.int32),
            pltpu.VMEM((per_s,), jnp.int32),
            pltpu.VMEM((per_s,), jnp.int32),
            pltpu.VMEM((per_s,), jnp.float32),
            pltpu.VMEM((per_s // 128, 128), jnp.int32),
            pltpu.VMEM((per_s // 128, 128), jnp.int32),
            pltpu.VMEM((per_s // 128, 128), jnp.float32),
            pltpu.VMEM((zchunk,), jnp.int32),
            pltpu.VMEM((zchunk,), jnp.float32),
            pltpu.VMEM((nbpad,), jnp.int32),
            pltpu.VMEM((L,), jnp.int32),
            pltpu.VMEM((g_per_w,), jnp.int32),
            pltpu.VMEM((gch, d), jnp.float32),
            pltpu.VMEM((gch, d), jnp.float32),
            pltpu.VMEM((gch, d), jnp.float32),
            pltpu.VMEM((gch, d), jnp.float32),
            pltpu.SemaphoreType.DMA,
            pltpu.SemaphoreType.DMA,
            pltpu.SemaphoreType.DMA,
            pltpu.SemaphoreType.DMA,
            pltpu.VMEM_SHARED((padn,), jnp.int32),
            pltpu.VMEM_SHARED((padn,), jnp.float32),
        ],
    )
    def dispatch(idx_hbm, rank_hbm, w_hbm, meta_hbm, x_hbm,
                 pos_hbm, dw_hbm, bexp_hbm, nblk_hbm, xg_hbm,
                 meta_v, a_idx, a_rank, posbuf,
                 b_idx, b_rank, b_w, sc_idx, sc_tok, sc_w,
                 zero_i, zero_f, bexp_v, nblk_v,
                 gidx_v, ga0, ga1, gb0, gb1, gsa0, gsa1, gsb0, gsb1,
                 sh_tok, sh_w):
        c = lax.axis_index("c")
        s = lax.axis_index("s")
        wid = s * NC + c
        lane = lax.iota(jnp.int32, L)

        pltpu.sync_copy(meta_hbm, meta_v)
        n_tok_mask = n_assign // 2 - 1
        off_row = meta_v[0]
        ends_row = meta_v[1]
        offs = [off_row[e] for e in range(n_e)]
        ends_sc = [ends_row[e] for e in range(n_e)]

        def sel_off(ev):
            acc = jnp.zeros((L,), jnp.int32)
            for e in range(n_e):
                acc = jnp.where(ev == e, offs[e], acc)
            return acc

        base = wid * per_w
        pltpu.sync_copy(idx_hbm.at[pl.ds(base, per_w)], a_idx)
        pltpu.sync_copy(rank_hbm.at[pl.ds(base, per_w)], a_rank)
        for k in range(per_w // L):
            ev = a_idx[pl.ds(k * L, L)]
            posbuf[pl.ds(k * L, L)] = a_rank[pl.ds(k * L, L)] + sel_off(ev)
        pltpu.sync_copy(posbuf, pos_hbm.at[pl.ds(base, per_w)])

        for k in range(zchunk // L):
            zero_i[pl.ds(k * L, L)] = jnp.zeros((L,), jnp.int32)
            zero_f[pl.ds(k * L, L)] = jnp.zeros((L,), jnp.float32)
        pltpu.sync_copy(zero_i, sh_tok.at[pl.ds(s * zchunk, zchunk)])
        pltpu.sync_copy(zero_f, sh_w.at[pl.ds(s * zchunk, zchunk)])
        plsc.subcore_barrier()

        sbase = s * per_s
        pltpu.sync_copy(idx_hbm.at[pl.ds(sbase, per_s)], b_idx)
        pltpu.sync_copy(rank_hbm.at[pl.ds(sbase, per_s)], b_rank)
        pltpu.sync_copy(w_hbm.at[pl.ds(sbase, per_s)], b_w)
        for q in range(per_s // 128):
            for k in range(128 // L):
                j0 = q * 128 + k * L
                ev = b_idx[pl.ds(j0, L)]
                sc_idx[q, pl.ds(k * L, L)] = b_rank[pl.ds(j0, L)] + sel_off(ev)
                sc_tok[q, pl.ds(k * L, L)] = (sbase + j0 + lane) & n_tok_mask
                sc_w[q, pl.ds(k * L, L)] = b_w[pl.ds(j0, L)]

        for q in range(per_s // 128):
            pltpu.sync_copy(sc_tok.at[q], sh_tok.at[sc_idx.at[q]], add=True)

        @pl.when(c == 1)
        def _():
            for q in range(per_s // 128):
                pltpu.sync_copy(sc_w.at[q], sh_w.at[sc_idx.at[q]], add=True)

        plsc.subcore_barrier()

        @pl.when((s == 0) & (c == 1))
        def _():
            pltpu.sync_copy(sh_w, dw_hbm)

        @pl.when((s == 1) & (c == 0))
        def _():
            for ch in range(nb_chunks):
                bstart = (ch * L + lane) * TB
                acc = jnp.zeros((L,), jnp.int32)
                for e in range(n_e):
                    acc = acc + jnp.where(bstart >= ends_sc[e], 1, 0)
                bexp_v[pl.ds(ch * L, L)] = jnp.minimum(acc, n_e - 1)
            pltpu.sync_copy(bexp_v, bexp_hbm)
            tb_sh = TB.bit_length() - 1
            nreal = (ends_sc[n_e - 1] + (TB - 1)) >> tb_sh
            nblk_v[...] = jnp.where(lane == 0, nreal, 0)
            pltpu.sync_copy(nblk_v, nblk_hbm)

        gbase = wid * g_per_w
        pltpu.sync_copy(sh_tok.at[pl.ds(gbase, g_per_w)], gidx_v)
        abuf = (ga0, ga1)
        bbuf = (gb0, gb1)
        asem = (gsa0, gsa1)
        bsem = (gsb0, gsb1)

        def fire_a(k):
            return pltpu.async_copy(
                x_hbm.at[gidx_v.at[pl.ds(k * gch, gch)]],
                abuf[k % 2], asem[k % 2])

        def fire_b(k):
            return pltpu.async_copy(
                x_hbm.at[gidx_v.at[pl.ds(ghalf + k * gch, gch)]],
                bbuf[k % 2], bsem[k % 2])

        acp = [None] * ngch
        bcp = [None] * ngch
        acp[0] = fire_a(0)
        bcp[0] = fire_b(0)
        for k in range(ngch):
            acp[k].wait()
            if k + 1 < ngch:
                acp[k + 1] = fire_a(k + 1)
            pltpu.sync_copy(abuf[k % 2], xg_hbm.at[pl.ds(gbase + k * gch, gch)])
            bcp[k].wait()
            if k + 1 < ngch:
                bcp[k + 1] = fire_b(k + 1)
            pltpu.sync_copy(bbuf[k % 2],
                            xg_hbm.at[pl.ds(gbase + ghalf + k * gch, gch)])

    return dispatch


def _expert_body_first(bexp_ref, nblk_ref, xg_ref, wg_ref, wu_ref, wd_ref,
                       dw_ref, od_ref):
    @pl.when(pl.program_id(0) < nblk_ref[0])
    def _():
        xb = xg_ref[...].astype(jnp.float32)
        g = jnp.dot(xb, wg_ref[0], preferred_element_type=jnp.float32)
        u = jnp.dot(xb, wu_ref[0], preferred_element_type=jnp.float32)
        h = g * jax.nn.sigmoid(g) * u
        part = jnp.dot(h, wd_ref[0], preferred_element_type=jnp.float32) * dw_ref[...]
        od_ref[...] = part.astype(od_ref.dtype)


def _expert_body_second(bexp_ref, nblk_ref, xg_ref, wg_ref, wu_ref, wd_ref,
                        dw_ref, acc_ref, od_ref):
    @pl.when(pl.program_id(0) < nblk_ref[0])
    def _():
        xb = xg_ref[...].astype(jnp.float32)
        g = jnp.dot(xb, wg_ref[0], preferred_element_type=jnp.float32)
        u = jnp.dot(xb, wu_ref[0], preferred_element_type=jnp.float32)
        h = g * jax.nn.sigmoid(g) * u
        part = jnp.dot(h, wd_ref[0], preferred_element_type=jnp.float32) * dw_ref[...]
        od_ref[...] = (acc_ref[...].astype(jnp.float32) + part).astype(od_ref.dtype)


def _experts(bexp, nblk, xg, W_gate, W_up, W_down, dw2d, nb, od_dt):
    n_e, d, ff = W_gate.shape
    padn = xg.shape[0]
    fhalf = ff // 2

    def specs(fi, extra):
        return [
            pl.BlockSpec((TB, d), lambda b, be, nr: (b, 0)),
            pl.BlockSpec((1, d, fhalf), lambda b, be, nr: (be[b], 0, fi)),
            pl.BlockSpec((1, d, fhalf), lambda b, be, nr: (be[b], 0, fi)),
            pl.BlockSpec((1, fhalf, d), lambda b, be, nr: (be[b], fi, 0)),
            pl.BlockSpec((TB, 1), lambda b, be, nr: (b, 0)),
        ] + extra

    od_shape = jax.ShapeDtypeStruct((padn, d), od_dt)
    out_spec = pl.BlockSpec((TB, d), lambda b, be, nr: (b, 0))
    params = pltpu.CompilerParams(dimension_semantics=("arbitrary",))

    acc = pl.pallas_call(
        _expert_body_first,
        grid_spec=pltpu.PrefetchScalarGridSpec(
            num_scalar_prefetch=2, grid=(nb,),
            in_specs=specs(0, []), out_specs=out_spec),
        out_shape=od_shape,
        compiler_params=params,
    )(bexp, nblk, xg, W_gate, W_up, W_down, dw2d)

    return pl.pallas_call(
        _expert_body_second,
        grid_spec=pltpu.PrefetchScalarGridSpec(
            num_scalar_prefetch=2, grid=(nb,),
            in_specs=specs(1, [pl.BlockSpec((TB, d), lambda b, be, nr: (b, 0))]),
            out_specs=out_spec),
        out_shape=od_shape,
        compiler_params=params,
    )(bexp, nblk, xg, W_gate, W_up, W_down, dw2d, acc)


def _make_combine(n_tok, padn, d, dt):
    tok_per_w = n_tok // NW
    ch = 16
    nch = tok_per_w // ch
    lanes = 32 if jnp.dtype(dt).itemsize == 2 else 16
    mesh = plsc.VectorSubcoreMesh(core_axis_name="c", subcore_axis_name="s")

    @functools.partial(
        pl.kernel,
        out_type=jax.ShapeDtypeStruct((n_tok, d), dt),
        mesh=mesh,
        scratch_types=[
            pltpu.VMEM((tok_per_w,), jnp.int32),
            pltpu.VMEM((tok_per_w,), jnp.int32),
            pltpu.VMEM((ch, d), dt),
            pltpu.VMEM((ch, d), dt),
            pltpu.VMEM((ch, d), dt),
            pltpu.VMEM((ch, d), dt),
            pltpu.SemaphoreType.DMA,
            pltpu.SemaphoreType.DMA,
            pltpu.SemaphoreType.DMA,
            pltpu.SemaphoreType.DMA,
        ],
    )
    def combine(od_hbm, pos_hbm, out_hbm,
                pos0_v, pos1_v, a0, b0, a1, b1, sa0, sb0, sa1, sb1):
        c = lax.axis_index("c")
        s = lax.axis_index("s")
        wid = s * NC + c
        tbase = wid * tok_per_w
        n_tok_half = pos_hbm.shape[0] // 2
        n_col = d // lanes
        pltpu.sync_copy(pos_hbm.at[pl.ds(tbase, tok_per_w)], pos0_v)
        pltpu.sync_copy(pos_hbm.at[pl.ds(n_tok_half + tbase, tok_per_w)], pos1_v)
        abuf = (a0, a1)
        bbuf = (b0, b1)
        asem = (sa0, sa1)
        bsem = (sb0, sb1)

        def fire(k):
            p = k % 2
            return (pltpu.async_copy(
                        od_hbm.at[pos0_v.at[pl.ds(k * ch, ch)]], abuf[p], asem[p]),
                    pltpu.async_copy(
                        od_hbm.at[pos1_v.at[pl.ds(k * ch, ch)]], bbuf[p], bsem[p]))

        cps = [None] * nch
        cps[0] = fire(0)
        for k in range(nch):
            p = k % 2
            cps[k][0].wait()
            cps[k][1].wait()
            if k + 1 < nch:
                cps[k + 1] = fire(k + 1)
            ra, rb = abuf[p], bbuf[p]

            def addb(i, carry):
                r = i // n_col
                col = (i % n_col) * lanes
                ra[r, pl.ds(col, lanes)] = (ra[r, pl.ds(col, lanes)]
                                            + rb[r, pl.ds(col, lanes)])
                return carry

            lax.fori_loop(0, ch * n_col, addb, 0)
            pltpu.sync_copy(ra, out_hbm.at[pl.ds(tbase + k * ch, ch)])

    return combine


def kernel_routed(x, W_router, W_gate, W_up, W_down):
    b, t, d = x.shape
    n_tok = b * t
    n_e = W_router.shape[1]
    n_assign = 2 * n_tok
    padn = n_assign + n_e * TB
    nb = padn // TB
    nbpad = ((nb + L - 1) // L) * L
    x_flat = x.reshape(n_tok, d)

    scores, weights, indices, rank, meta = _router(x_flat, W_router)
    idx8 = indices.T.reshape(n_assign)
    rank8 = rank.T.reshape(n_assign)
    w8 = weights.T.reshape(n_assign)

    pos, dw, bexp, nblk, xg = _make_dispatch(n_assign, padn, nbpad, n_e, d)(
        idx8, rank8, w8, meta, x_flat)
    od = _experts(bexp, nblk, xg, W_gate, W_up, W_down,
                  dw.reshape(padn, 1), nb, jnp.float32)
    out_flat = _make_combine(n_tok, padn, d, jnp.float32)(od, pos)
    return out_flat.reshape(b, t, d), weights, indices, scores


kernel = kernel_routed


# speedup vs baseline: 1.7102x; 1.1221x over previous
import functools

import jax
import jax.numpy as jnp
from jax import lax
from jax.experimental import pallas as pl
from jax.experimental.pallas import tpu as pltpu
from jax.experimental.pallas import tpu_sc as plsc

TB = 256
NC, NS, L = 2, 16, 16
NW = NC * NS


def _router_body(x_ref, wr_ref, scores_ref, w_ref, i_ref, rank_ref,
                 counts_ref, run_ref, *, n_e, n_t):
    t = pl.program_id(0)
    tb = x_ref.shape[0]
    logits = jnp.dot(x_ref[...], wr_ref[...], preferred_element_type=jnp.float32)
    m = jnp.max(logits, axis=-1, keepdims=True)
    ex = jnp.exp(logits - m)
    sc = ex / jnp.sum(ex, axis=-1, keepdims=True)
    scores_ref[...] = sc
    eio = lax.broadcasted_iota(jnp.int32, sc.shape, 1)
    w0 = jnp.max(sc, axis=-1, keepdims=True)
    i0 = jnp.min(jnp.where(sc == w0, eio, n_e), axis=-1, keepdims=True)
    sc2 = jnp.where(eio == i0, -1.0, sc)
    w1 = jnp.max(sc2, axis=-1, keepdims=True)
    i1 = jnp.min(jnp.where(sc2 == w1, eio, n_e), axis=-1, keepdims=True)
    ssum = w0 + w1
    w0n = w0 / ssum
    w1n = w1 / ssum
    lane2 = lax.broadcasted_iota(jnp.int32, (tb, 2), 1)
    w_ref[...] = jnp.where(lane2 == 0, w0n, w1n)
    i_ref[...] = jnp.where(lane2 == 0, i0, i1)
    oh0 = (eio == i0).astype(jnp.float32)
    oh1 = (eio == i1).astype(jnp.float32)
    oh = oh0 + oh1
    r_iota = lax.broadcasted_iota(jnp.int32, (tb, tb), 0)
    c_iota = lax.broadcasted_iota(jnp.int32, (tb, tb), 1)
    tri = (c_iota < r_iota).astype(jnp.float32)
    base = jnp.where(t == 0, jnp.zeros_like(run_ref), run_ref[...])
    cum = base + jnp.dot(tri, oh, preferred_element_type=jnp.float32)
    rank0 = jnp.sum(oh0 * cum, axis=1, keepdims=True)
    rank1 = jnp.sum(oh1 * cum, axis=1, keepdims=True)
    rank_ref[...] = jnp.where(lane2 == 0, rank0, rank1).astype(jnp.int32)
    newrun = base + jnp.sum(oh, axis=0, keepdims=True)
    run_ref[...] = newrun

    @pl.when(t == n_t - 1)
    def _():
        cnt16 = jnp.concatenate(
            [newrun, jnp.zeros((1, L - n_e), jnp.float32)], axis=1)
        padded = jnp.floor((cnt16 + (TB - 1)) / TB) * TB
        a16 = lax.broadcasted_iota(jnp.int32, (L, L), 0)
        b16 = lax.broadcasted_iota(jnp.int32, (L, L), 1)
        tri_incl = (a16 <= b16).astype(jnp.float32)
        ends = jnp.dot(padded, tri_incl, preferred_element_type=jnp.float32)
        off = ends - padded
        counts_ref[...] = jnp.concatenate([off, ends], axis=0).astype(jnp.int32)


def _router(x_flat, W_router):
    n_tok, d = x_flat.shape
    n_e = W_router.shape[1]
    n_t = n_tok // TB
    return pl.pallas_call(
        functools.partial(_router_body, n_e=n_e, n_t=n_t),
        grid=(n_t,),
        in_specs=[pl.BlockSpec((TB, d), lambda i: (i, 0)),
                  pl.BlockSpec((d, n_e), lambda i: (0, 0))],
        out_specs=[pl.BlockSpec((TB, n_e), lambda i: (i, 0)),
                   pl.BlockSpec((TB, 2), lambda i: (i, 0)),
                   pl.BlockSpec((TB, 2), lambda i: (i, 0)),
                   pl.BlockSpec((TB, 2), lambda i: (i, 0)),
                   pl.BlockSpec((2, L), lambda i: (0, 0))],
        out_shape=[jax.ShapeDtypeStruct((n_tok, n_e), jnp.float32),
                   jax.ShapeDtypeStruct((n_tok, 2), jnp.float32),
                   jax.ShapeDtypeStruct((n_tok, 2), jnp.int32),
                   jax.ShapeDtypeStruct((n_tok, 2), jnp.int32),
                   jax.ShapeDtypeStruct((2, L), jnp.int32)],
        scratch_shapes=[pltpu.VMEM((1, n_e), jnp.float32)],
        compiler_params=pltpu.CompilerParams(dimension_semantics=("arbitrary",)),
    )(x_flat, W_router)


def _make_dispatch(n_assign, padn, nbpad, n_e, d):
    per_w = n_assign // NW
    per_s = n_assign // NS
    zchunk = padn // NS
    nb_chunks = nbpad // L
    g_per_w = padn // NW
    ghalf = g_per_w // 2
    gch = 64
    ngch = ghalf // gch
    mesh = plsc.VectorSubcoreMesh(core_axis_name="c", subcore_axis_name="s")

    @functools.partial(
        pl.kernel,
        out_type=[jax.ShapeDtypeStruct((n_assign,), jnp.int32),
                  jax.ShapeDtypeStruct((padn,), jnp.float32),
                  jax.ShapeDtypeStruct((nbpad,), jnp.int32),
                  jax.ShapeDtypeStruct((L,), jnp.int32),
                  jax.ShapeDtypeStruct((padn, d), jnp.float32)],
        mesh=mesh,
        scratch_types=[
            pltpu.VMEM((2, L), jnp.int32),
            pltpu.VMEM((per_w,), jnp.int32),
            pltpu.VMEM((per_w,), jnp.int32),
            pltpu.VMEM((per_w,), jnp.int32),
            pltpu.VMEM((per_s,), jnp.int32),
            pltpu.VMEM((per_s,), jnp.int32),
            pltpu.VMEM((per_s,), jnp.float32),
            pltpu.VMEM((per_s // 128, 128), jnp.int32),
            pltpu.VMEM((per_s // 128, 128), jnp.int32),
            pltpu.VMEM((per_s // 128, 128), jnp.float32),
            pltpu.VMEM((zchunk,), jnp.int32),
            pltpu.VMEM((zchunk,), jnp.float32),
            pltpu.VMEM((nbpad,), jnp.int32),
            pltpu.VMEM((L,), jnp.int32),
            pltpu.VMEM((g_per_w,), jnp.int32),
            pltpu.VMEM((gch, d), jnp.float32),
            pltpu.SemaphoreType.DMA,
            pltpu.VMEM_SHARED((padn,), jnp.int32),
            pltpu.VMEM_SHARED((padn,), jnp.float32),
        ],
    )
    def dispatch(idx_hbm, rank_hbm, w_hbm, meta_hbm, x_hbm,
                 pos_hbm, dw_hbm, bexp_hbm, nblk_hbm, xg_hbm,
                 meta_v, a_idx, a_rank, posbuf,
                 b_idx, b_rank, b_w, sc_idx, sc_tok, sc_w,
                 zero_i, zero_f, bexp_v, nblk_v,
                 gidx_v, ga0, gsa0, sh_tok, sh_w):
        c = lax.axis_index("c")
        s = lax.axis_index("s")
        wid = s * NC + c
        lane = lax.iota(jnp.int32, L)

        pltpu.sync_copy(meta_hbm, meta_v)
        n_tok_mask = n_assign // 2 - 1
        off_row = meta_v[0]
        ends_row = meta_v[1]
        offs = [off_row[e] for e in range(n_e)]
        ends_sc = [ends_row[e] for e in range(n_e)]

        def sel_off(ev):
            acc = jnp.zeros((L,), jnp.int32)
            for e in range(n_e):
                acc = jnp.where(ev == e, offs[e], acc)
            return acc

        base = wid * per_w
        pltpu.sync_copy(idx_hbm.at[pl.ds(base, per_w)], a_idx)
        pltpu.sync_copy(rank_hbm.at[pl.ds(base, per_w)], a_rank)
        for k in range(per_w // L):
            ev = a_idx[pl.ds(k * L, L)]
            posbuf[pl.ds(k * L, L)] = a_rank[pl.ds(k * L, L)] + sel_off(ev)
        pltpu.sync_copy(posbuf, pos_hbm.at[pl.ds(base, per_w)])

        for k in range(zchunk // L):
            zero_i[pl.ds(k * L, L)] = jnp.zeros((L,), jnp.int32)
            zero_f[pl.ds(k * L, L)] = jnp.zeros((L,), jnp.float32)
        pltpu.sync_copy(zero_i, sh_tok.at[pl.ds(s * zchunk, zchunk)])
        pltpu.sync_copy(zero_f, sh_w.at[pl.ds(s * zchunk, zchunk)])
        plsc.subcore_barrier()

        sbase = s * per_s
        pltpu.sync_copy(idx_hbm.at[pl.ds(sbase, per_s)], b_idx)
        pltpu.sync_copy(rank_hbm.at[pl.ds(sbase, per_s)], b_rank)
        pltpu.sync_copy(w_hbm.at[pl.ds(sbase, per_s)], b_w)
        for q in range(per_s // 128):
            for k in range(128 // L):
                j0 = q * 128 + k * L
                ev = b_idx[pl.ds(j0, L)]
                sc_idx[q, pl.ds(k * L, L)] = b_rank[pl.ds(j0, L)] + sel_off(ev)
                sc_tok[q, pl.ds(k * L, L)] = (sbase + j0 + lane) & n_tok_mask
                sc_w[q, pl.ds(k * L, L)] = b_w[pl.ds(j0, L)]

        for q in range(per_s // 128):
            pltpu.sync_copy(sc_tok.at[q], sh_tok.at[sc_idx.at[q]], add=True)

        @pl.when(c == 1)
        def _():
            for q in range(per_s // 128):
                pltpu.sync_copy(sc_w.at[q], sh_w.at[sc_idx.at[q]], add=True)

        plsc.subcore_barrier()

        @pl.when((s == 0) & (c == 1))
        def _():
            pltpu.sync_copy(sh_w, dw_hbm)

        @pl.when((s == 1) & (c == 0))
        def _():
            for ch in range(nb_chunks):
                bstart = (ch * L + lane) * TB
                acc = jnp.zeros((L,), jnp.int32)
                for e in range(n_e):
                    acc = acc + jnp.where(bstart >= ends_sc[e], 1, 0)
                bexp_v[pl.ds(ch * L, L)] = jnp.minimum(acc, n_e - 1)
            pltpu.sync_copy(bexp_v, bexp_hbm)
            tb_sh = TB.bit_length() - 1
            nreal = (ends_sc[n_e - 1] + (TB - 1)) >> tb_sh
            nblk_v[...] = jnp.where(lane == 0, nreal, 0)
            pltpu.sync_copy(nblk_v, nblk_hbm)

        gbase = wid * g_per_w
        pltpu.sync_copy(sh_tok.at[pl.ds(gbase, g_per_w)], gidx_v)
        gch_sh = gch.bit_length() - 1
        ptotal = ends_sc[n_e - 1]
        remaining = jnp.maximum(ptotal - gbase, 0)
        nd = jnp.minimum((remaining + (gch - 1)) >> gch_sh, g_per_w // gch)

        def gbody(k, carry):
            cp = pltpu.async_copy(
                x_hbm.at[gidx_v.at[pl.ds(k * gch, gch)]], ga0, gsa0)
            cp.wait()
            pltpu.sync_copy(ga0, xg_hbm.at[pl.ds(gbase + k * gch, gch)])
            return carry

        lax.fori_loop(0, nd, gbody, 0)

    return dispatch


def _make_gather(padn, d, dt):
    per_w = padn // NW
    ch = 40 if jnp.dtype(dt).itemsize == 4 else 64
    nch = per_w // ch
    mesh = plsc.VectorSubcoreMesh(core_axis_name="c", subcore_axis_name="s")

    @functools.partial(
        pl.kernel,
        out_type=jax.ShapeDtypeStruct((padn, d), dt),
        mesh=mesh,
        scratch_types=[
            pltpu.VMEM((per_w,), jnp.int32),
            pltpu.VMEM((ch, d), dt),
            pltpu.VMEM((ch, d), dt),
            pltpu.SemaphoreType.DMA,
            pltpu.SemaphoreType.DMA,
        ],
    )
    def gather(x_hbm, dtok_hbm, xg_hbm, idx_v, rows0, rows1, sem0, sem1):
        c = lax.axis_index("c")
        s = lax.axis_index("s")
        base = (s * NC + c) * per_w
        pltpu.sync_copy(dtok_hbm.at[pl.ds(base, per_w)], idx_v)
        bufs = (rows0, rows1)
        sems = (sem0, sem1)
        cps = [None] * nch
        cps[0] = pltpu.async_copy(
            x_hbm.at[idx_v.at[pl.ds(0, ch)]], bufs[0], sems[0])
        for k in range(nch):
            cps[k].wait()
            if k + 1 < nch:
                cps[k + 1] = pltpu.async_copy(
                    x_hbm.at[idx_v.at[pl.ds((k + 1) * ch, ch)]],
                    bufs[(k + 1) % 2], sems[(k + 1) % 2])
            pltpu.sync_copy(bufs[k % 2], xg_hbm.at[pl.ds(base + k * ch, ch)])

    return gather


def _expert_body_first(bexp_ref, nblk_ref, xg_ref, wg_ref, wu_ref, wd_ref,
                       dw_ref, od_ref):
    @pl.when(pl.program_id(0) < nblk_ref[0])
    def _():
        xb = xg_ref[...].astype(jnp.float32)
        g = jnp.dot(xb, wg_ref[0], preferred_element_type=jnp.float32)
        u = jnp.dot(xb, wu_ref[0], preferred_element_type=jnp.float32)
        h = g * jax.nn.sigmoid(g) * u
        part = jnp.dot(h, wd_ref[0], preferred_element_type=jnp.float32) * dw_ref[...]
        od_ref[...] = part.astype(od_ref.dtype)


def _expert_body_second(bexp_ref, nblk_ref, xg_ref, wg_ref, wu_ref, wd_ref,
                        dw_ref, acc_ref, od_ref):
    @pl.when(pl.program_id(0) < nblk_ref[0])
    def _():
        xb = xg_ref[...].astype(jnp.float32)
        g = jnp.dot(xb, wg_ref[0], preferred_element_type=jnp.float32)
        u = jnp.dot(xb, wu_ref[0], preferred_element_type=jnp.float32)
        h = g * jax.nn.sigmoid(g) * u
        part = jnp.dot(h, wd_ref[0], preferred_element_type=jnp.float32) * dw_ref[...]
        od_ref[...] = (acc_ref[...].astype(jnp.float32) + part).astype(od_ref.dtype)


def _experts(bexp, nblk, xg, W_gate, W_up, W_down, dw2d, nb, od_dt):
    n_e, d, ff = W_gate.shape
    padn = xg.shape[0]
    fhalf = ff // 2

    def specs(fi, extra):
        return [
            pl.BlockSpec((TB, d), lambda b, be, nr: (b, 0)),
            pl.BlockSpec((1, d, fhalf), lambda b, be, nr: (be[b], 0, fi)),
            pl.BlockSpec((1, d, fhalf), lambda b, be, nr: (be[b], 0, fi)),
            pl.BlockSpec((1, fhalf, d), lambda b, be, nr: (be[b], fi, 0)),
            pl.BlockSpec((TB, 1), lambda b, be, nr: (b, 0)),
        ] + extra

    od_shape = jax.ShapeDtypeStruct((padn, d), od_dt)
    out_spec = pl.BlockSpec((TB, d), lambda b, be, nr: (b, 0))
    params = pltpu.CompilerParams(dimension_semantics=("arbitrary",))

    acc = pl.pallas_call(
        _expert_body_first,
        grid_spec=pltpu.PrefetchScalarGridSpec(
            num_scalar_prefetch=2, grid=(nb,),
            in_specs=specs(0, []), out_specs=out_spec),
        out_shape=od_shape,
        compiler_params=params,
    )(bexp, nblk, xg, W_gate, W_up, W_down, dw2d)

    return pl.pallas_call(
        _expert_body_second,
        grid_spec=pltpu.PrefetchScalarGridSpec(
            num_scalar_prefetch=2, grid=(nb,),
            in_specs=specs(1, [pl.BlockSpec((TB, d), lambda b, be, nr: (b, 0))]),
            out_specs=out_spec),
        out_shape=od_shape,
        compiler_params=params,
    )(bexp, nblk, xg, W_gate, W_up, W_down, dw2d, acc)


def _make_combine(n_tok, padn, d, dt):
    tok_per_w = n_tok // NW
    ch = 16
    nch = tok_per_w // ch
    lanes = 32 if jnp.dtype(dt).itemsize == 2 else 16
    mesh = plsc.VectorSubcoreMesh(core_axis_name="c", subcore_axis_name="s")

    @functools.partial(
        pl.kernel,
        out_type=jax.ShapeDtypeStruct((n_tok, d), dt),
        mesh=mesh,
        scratch_types=[
            pltpu.VMEM((tok_per_w,), jnp.int32),
            pltpu.VMEM((tok_per_w,), jnp.int32),
            pltpu.VMEM((ch, d), dt),
            pltpu.VMEM((ch, d), dt),
            pltpu.VMEM((ch, d), dt),
            pltpu.VMEM((ch, d), dt),
            pltpu.SemaphoreType.DMA,
            pltpu.SemaphoreType.DMA,
            pltpu.SemaphoreType.DMA,
            pltpu.SemaphoreType.DMA,
        ],
    )
    def combine(od_hbm, pos_hbm, out_hbm,
                pos0_v, pos1_v, a0, b0, a1, b1, sa0, sb0, sa1, sb1):
        c = lax.axis_index("c")
        s = lax.axis_index("s")
        wid = s * NC + c
        tbase = wid * tok_per_w
        n_tok_half = pos_hbm.shape[0] // 2
        n_col = d // lanes
        pltpu.sync_copy(pos_hbm.at[pl.ds(tbase, tok_per_w)], pos0_v)
        pltpu.sync_copy(pos_hbm.at[pl.ds(n_tok_half + tbase, tok_per_w)], pos1_v)
        abuf = (a0, a1)
        bbuf = (b0, b1)
        asem = (sa0, sa1)
        bsem = (sb0, sb1)

        def fire(k):
            p = k % 2
            return (pltpu.async_copy(
                        od_hbm.at[pos0_v.at[pl.ds(k * ch, ch)]], abuf[p], asem[p]),
                    pltpu.async_copy(
                        od_hbm.at[pos1_v.at[pl.ds(k * ch, ch)]], bbuf[p], bsem[p]))

        cps = [None] * nch
        cps[0] = fire(0)
        for k in range(nch):
            p = k % 2
            cps[k][0].wait()
            cps[k][1].wait()
            if k + 1 < nch:
                cps[k + 1] = fire(k + 1)
            ra, rb = abuf[p], bbuf[p]

            def addb(i, carry):
                r = i // n_col
                col = (i % n_col) * lanes
                ra[r, pl.ds(col, lanes)] = (ra[r, pl.ds(col, lanes)]
                                            + rb[r, pl.ds(col, lanes)])
                return carry

            lax.fori_loop(0, ch * n_col, addb, 0)
            pltpu.sync_copy(ra, out_hbm.at[pl.ds(tbase + k * ch, ch)])

    return combine


def kernel_routed(x, W_router, W_gate, W_up, W_down):
    b, t, d = x.shape
    n_tok = b * t
    n_e = W_router.shape[1]
    n_assign = 2 * n_tok
    padn = n_assign + n_e * TB
    nb = padn // TB
    nbpad = ((nb + L - 1) // L) * L
    x_flat = x.reshape(n_tok, d)

    scores, weights, indices, rank, meta = _router(x_flat, W_router)
    idx8 = indices.T.reshape(n_assign)
    rank8 = rank.T.reshape(n_assign)
    w8 = weights.T.reshape(n_assign)

    pos, dw, bexp, nblk, xg = _make_dispatch(n_assign, padn, nbpad, n_e, d)(
        idx8, rank8, w8, meta, x_flat)
    od = _experts(bexp, nblk, xg, W_gate, W_up, W_down,
                  dw.reshape(padn, 1), nb, jnp.float32)
    out_flat = _make_combine(n_tok, padn, d, jnp.float32)(od, pos)
    return out_flat.reshape(b, t, d), weights, indices, scores


kernel = kernel_routed
